# baseline (speedup 1.0000x reference)
import functools

import jax
import jax.numpy as jnp
from jax import lax
from jax.experimental import pallas as pl
from jax.experimental.pallas import tpu as pltpu
from jax.experimental.pallas import tpu_sc as plsc

N = 50000
E = 800000
ROWS = E // 128
NSC = 2
NTILE = 16
NROW = N // NTILE
R_TC = 5000
GRID = N // R_TC

_MESH = dict(core_axis_name="c", subcore_axis_name="s")


def _deg_body(dst_hbm, out_hbm, idx_v, part_v, ones_v):
    cid = lax.axis_index("c")
    sid = lax.axis_index("s")
    wid = sid * NSC + cid
    per = E // (NSC * NTILE)
    nfull = per // 16
    rem = per - nfull * 16

    def zbody(i, _):
        part_v[pl.ds(i * 16, 16)] = jnp.zeros((16,), jnp.float32)
        return _
    lax.fori_loop(0, N // 16, zbody, 0)
    idx_v[pl.ds(per, 16)] = jnp.zeros((16,), jnp.int32)
    pltpu.sync_copy(dst_hbm.at[pl.ds(wid * per, per)], idx_v.at[pl.ds(0, per)])

    def body(i, _):
        iv = idx_v[pl.ds(i * 16, 16)]
        plsc.addupdate_scatter(part_v, [iv], ones_v[...])
        return _
    lax.fori_loop(0, nfull, body, 0)
    iv = idx_v[pl.ds(nfull * 16, 16)]
    mask = lax.iota(jnp.int32, 16) < rem
    plsc.addupdate_scatter(part_v, [iv], ones_v[...], mask=mask)
    pltpu.sync_copy(part_v, out_hbm.at[wid])


@functools.partial(
    pl.kernel,
    out_type=jax.ShapeDtypeStruct((NSC * NTILE, N), jnp.float32),
    mesh=plsc.VectorSubcoreMesh(**_MESH),
    compiler_params=pltpu.CompilerParams(needs_layout_passes=False, use_tc_tiling_on_sc=False),
    scratch_types=[
        pltpu.VMEM((E // (NSC * NTILE) + 16,), jnp.int32),
        pltpu.VMEM((N,), jnp.float32),
        pltpu.VMEM((16,), jnp.float32),
    ],
)
def _deg_kernel(dst_hbm, out_hbm, idx_v, part_v, ones_v):
    ones_v[...] = jnp.ones((16,), jnp.float32)
    _deg_body(dst_hbm, out_hbm, idx_v, part_v, ones_v)


def _make_prop(D, split_edges):
    C = 64 if D == 16 else 16
    NB = 10 if D == 16 else 6

    def body(tab_hbm, src_hbm, dst_hbm, out_hbm, *refs):
        src_v, dst_v, srow_v, drow_v = refs[:4]
        rbufs = refs[4:4 + NB]
        acc_sh = refs[4 + NB]
        sems = refs[5 + NB:]
        zbuf = rbufs[0]
        cid = lax.axis_index("c")
        sid = lax.axis_index("s")

        def zb(i, _):
            for k in range(D // 16):
                zbuf[i, pl.ds(k * 16, 16)] = jnp.zeros((16,), jnp.float32)
            return _
        lax.fori_loop(0, 128, zb, 0)
        zbuf2d = zbuf
        base = sid * NROW
        nz = NROW // 128
        rz = NROW - nz * 128

        def zcopy(i, _):
            pltpu.sync_copy(zbuf2d, acc_sh.at[pl.ds(base + i * 128, 128)])
            return _
        lax.fori_loop(0, nz, zcopy, 0)
        if rz:
            pltpu.sync_copy(zbuf2d.at[pl.ds(0, rz)],
                            acc_sh.at[pl.ds(base + nz * 128, rz)])
        plsc.subcore_barrier()

        if split_edges:
            rows_sc = ROWS // NSC
            base_sc = cid * rows_sc
        else:
            rows_sc = ROWS
            base_sc = jnp.int32(0)
        q, r = rows_sc // NTILE, rows_sc % NTILE
        start = base_sc + sid * q + jnp.minimum(sid, r)
        count = q + (sid < r).astype(jnp.int32)

        nch = count // C
        rem = count - nch * C

        def chunk(g, _):
            r0 = start + g * C
            pltpu.sync_copy(src_hbm.at[pl.ds(r0, C)], src_v)
            pltpu.sync_copy(dst_hbm.at[pl.ds(r0, C)], dst_v)
            descs = [pltpu.async_copy(tab_hbm.at[cid].at[src_v.at[j]],
                                      rbufs[j], sems[j])
                     for j in range(NB)]
            for j in range(C):
                b = j % NB
                descs[b].wait()
                pltpu.sync_copy(rbufs[b], acc_sh.at[dst_v.at[j]], add=True)
                if j + NB < C:
                    descs[b] = pltpu.async_copy(
                        tab_hbm.at[cid].at[src_v.at[j + NB]], rbufs[b],
                        sems[b])
            return _
        lax.fori_loop(0, nch, chunk, 0)

        def tail(t, _):
            r0 = start + nch * C + t
            pltpu.sync_copy(src_hbm.at[r0], srow_v)
            pltpu.sync_copy(dst_hbm.at[r0], drow_v)
            pltpu.async_copy(tab_hbm.at[cid].at[srow_v], rbufs[0],
                             sems[0]).wait()
            pltpu.sync_copy(rbufs[0], acc_sh.at[drow_v], add=True)
            return _
        lax.fori_loop(0, rem, tail, 0)

        plsc.subcore_barrier()
        pltpu.sync_copy(acc_sh.at[pl.ds(base, NROW)],
                        out_hbm.at[cid, pl.ds(base, NROW)])

    kern = pl.kernel(
        body,
        out_type=jax.ShapeDtypeStruct((NSC, N, D), jnp.float32),
        mesh=plsc.VectorSubcoreMesh(**_MESH),
        compiler_params=pltpu.CompilerParams(needs_layout_passes=False, use_tc_tiling_on_sc=False),
        scratch_types=(
            [pltpu.VMEM((C, 128), jnp.int32),
             pltpu.VMEM((C, 128), jnp.int32),
             pltpu.VMEM((128,), jnp.int32),
             pltpu.VMEM((128,), jnp.int32)]
            + [pltpu.VMEM((128, D), jnp.float32) for _ in range(NB)]
            + [pltpu.VMEM_SHARED((N, D), jnp.float32)]
            + [pltpu.SemaphoreType.DMA for _ in range(NB)]
        ),
    )
    return kern


def _const(shape):
    return pl.BlockSpec(shape, lambda i: tuple(0 for _ in shape))


def _prep_body(degp, x, Wc1, bc1, Wc2, bc2, dis_o, s0_o, pc_o):
    deg = 1.0 + jnp.sum(degp[...], axis=1, keepdims=True)
    dis = lax.rsqrt(deg)
    dis_o[...] = dis
    xb = x[...]
    s = dis * xb
    s = jnp.pad(s, ((0, 0), (0, 6)))
    s0_o[0] = s
    s0_o[1] = s
    charges = xb[:, 8:9]
    h = jax.nn.relu(charges * Wc1[...] + bc1[...])
    pc_o[...] = jnp.dot(h, Wc2[...]) + bc2[...]


def _prep_call(deg_parts, x, Wc1, bc1, Wc2, bc2):
    return pl.pallas_call(
        _prep_body,
        grid=(GRID,),
        in_specs=[
            pl.BlockSpec((R_TC, NSC * NTILE), lambda i: (i, 0)),
            pl.BlockSpec((R_TC, 10), lambda i: (i, 0)),
            _const((1, 32)), _const((1, 32)),
            _const((32, 16)), _const((1, 16)),
        ],
        out_specs=[
            pl.BlockSpec((R_TC, 1), lambda i: (i, 0)),
            pl.BlockSpec((NSC, R_TC, 16), lambda i: (0, i, 0)),
            pl.BlockSpec((R_TC, 16), lambda i: (i, 0)),
        ],
        out_shape=[
            jax.ShapeDtypeStruct((N, 1), jnp.float32),
            jax.ShapeDtypeStruct((NSC, N, 16), jnp.float32),
            jax.ShapeDtypeStruct((N, 16), jnp.float32),
        ],
    )(deg_parts, x, Wc1, bc1, Wc2, bc2)


def _z1(acc, s0, dis, W1p, b1):
    p = dis[...] * (acc[0] + acc[1] + s0[0])
    return jnp.dot(p, W1p[...]) + b1[...]


def _z2(acc, s1, dis, b2):
    zl = dis[...] * (acc[0] + s1[0])
    zr = dis[...] * (acc[1] + s1[1])
    return jnp.concatenate([zl, zr], axis=1) + b2[...]


def _z3(acc, s2, dis, b3):
    return dis[...] * (acc[0] + acc[1] + s2[0]) + b3[...]


def _bn_relu(z, sums, g, b):
    mu = sums[0:1] * (1.0 / N)
    var = sums[1:2] * (1.0 / N) - mu * mu
    return jax.nn.relu(g * (z - mu) * lax.rsqrt(var + 1e-5) + b)


def _sums_body(zfn, nz_in, *refs):
    ins = refs[:nz_in]
    sums_o = refs[nz_in]
    accv = refs[nz_in + 1]
    z = zfn(*ins)
    i = pl.program_id(0)

    @pl.when(i == 0)
    def _():
        accv[...] = jnp.zeros_like(accv)

    accv[0:1] += jnp.sum(z, axis=0, keepdims=True)
    accv[1:2] += jnp.sum(z * z, axis=0, keepdims=True)

    @pl.when(i == GRID - 1)
    def _():
        sums_o[...] = accv[...]


def _layer_sums(zfn, nz_in, D, in_arrays, in_specs):
    return pl.pallas_call(
        functools.partial(_sums_body, zfn, nz_in),
        grid=(GRID,),
        in_specs=in_specs,
        out_specs=pl.BlockSpec((2, D), lambda i: (0, 0)),
        out_shape=jax.ShapeDtypeStruct((2, D), jnp.float32),
        scratch_shapes=[pltpu.VMEM((2, D), jnp.float32)],
    )(*in_arrays)


def kernel(x, edge_index, Wc1, bc1, Wc2, bc2, W1, b1, g1, be1,
           W2, b2, g2, be2, W3, b3, g3, be3, Wci, bci, Wcl, bcl):
    f32 = jnp.float32
    src2d = edge_index[0].reshape(ROWS, 128)
    dst2d = edge_index[1].reshape(ROWS, 128)
    W1p = jnp.pad(W1, ((0, 6), (0, 0)))
    r2 = lambda v: v.reshape(1, -1)
    bc1r, bc2r, b1r, g1r, be1r = map(r2, (bc1, bc2, b1, g1, be1))
    b2r, g2r, be2r, b3r, g3r, be3r = map(r2, (b2, g2, be2, b3, g3, be3))
    bcir, bclr = r2(bci), r2(bcl)

    deg_parts = _deg_kernel(edge_index[1]).T
    dis, s0, pc = _prep_call(deg_parts, x, Wc1.reshape(1, 32), bc1r,
                             Wc2, bc2r)

    prop16 = _make_prop(16, True)
    prop32e = _make_prop(32, True)
    prop32c = _make_prop(32, False)

    acc0 = prop16(s0, src2d, dst2d)

    spec_dis = pl.BlockSpec((R_TC, 1), lambda i: (i, 0))
    spec_p16 = pl.BlockSpec((NSC, R_TC, 16), lambda i: (0, i, 0))
    spec_p32 = pl.BlockSpec((NSC, R_TC, 32), lambda i: (0, i, 0))
    spec_s16 = pl.BlockSpec((1, R_TC, 16), lambda i: (0, i, 0))
    spec_s32 = pl.BlockSpec((1, R_TC, 32), lambda i: (0, i, 0))

    sums1 = _layer_sums(
        _z1, 5, 128,
        [acc0, s0, dis, W1p, b1r],
        [spec_p16, spec_s16, spec_dis, _const((16, 128)), _const((1, 128))])

    def _l1b(acc, s0r, disr, W1r, b1rr, sums, g, be, W2r, s1_o):
        z = _z1(acc, s0r, disr, W1r, b1rr)
        h = _bn_relu(z, sums[...], g[...], be[...])
        t = disr[...] * jnp.dot(h, W2r[...])
        s1_o[0] = t[:, :32]
        s1_o[1] = t[:, 32:]

    s1 = pl.pallas_call(
        _l1b,
        grid=(GRID,),
        in_specs=[spec_p16, spec_s16, spec_dis, _const((16, 128)),
                  _const((1, 128)), _const((2, 128)), _const((1, 128)),
                  _const((1, 128)), _const((128, 64))],
        out_specs=pl.BlockSpec((NSC, R_TC, 32), lambda i: (0, i, 0)),
        out_shape=jax.ShapeDtypeStruct((NSC, N, 32), f32),
    )(acc0, s0, dis, W1p, b1r, sums1, g1r, be1r, W2)

    acc1 = prop32c(s1, src2d, dst2d)

    sums2 = _layer_sums(
        _z2, 4, 64,
        [acc1, s1, dis, b2r],
        [spec_p32, spec_p32, spec_dis, _const((1, 64))])

    def _l2b(acc, s1r, disr, b2rr, sums, g, be, W3r, s2_o):
        z = _z2(acc, s1r, disr, b2rr)
        h = _bn_relu(z, sums[...], g[...], be[...])
        t = disr[...] * jnp.dot(h, W3r[...])
        s2_o[0] = t
        s2_o[1] = t

    s2 = pl.pallas_call(
        _l2b,
        grid=(GRID,),
        in_specs=[spec_p32, spec_p32, spec_dis, _const((1, 64)),
                  _const((2, 64)), _const((1, 64)), _const((1, 64)),
                  _const((64, 32))],
        out_specs=pl.BlockSpec((NSC, R_TC, 32), lambda i: (0, i, 0)),
        out_shape=jax.ShapeDtypeStruct((NSC, N, 32), f32),
    )(acc1, s1, dis, b2r, sums2, g2r, be2r, W3)

    acc2 = prop32e(s2, src2d, dst2d)

    sums3 = _layer_sums(
        _z3, 4, 32,
        [acc2, s2, dis, b3r],
        [spec_p32, spec_s32, spec_dis, _const((1, 32))])

    def _l3b(acc, s2r, disr, b3rr, sums, g, be, pcr, Wcir, bcirr, Wclr,
             bclrr, out_o):
        z = _z3(acc, s2r, disr, b3rr)
        h = _bn_relu(z, sums[...], g[...], be[...])
        hc = jnp.concatenate([h, pcr[...]], axis=1)
        hh = jax.nn.relu(jnp.dot(hc, Wcir[...]) + bcirr[...])
        out_o[...] = jax.nn.sigmoid(jnp.dot(hh, Wclr[...]) + bclrr[...])

    out = pl.pallas_call(
        _l3b,
        grid=(GRID,),
        in_specs=[spec_p32, spec_s32, spec_dis, _const((1, 32)),
                  _const((2, 32)), _const((1, 32)), _const((1, 32)),
                  pl.BlockSpec((R_TC, 16), lambda i: (i, 0)),
                  _const((48, 32)), _const((1, 32)),
                  _const((32, 1)), _const((1, 1))],
        out_specs=pl.BlockSpec((R_TC, 1), lambda i: (i, 0)),
        out_shape=jax.ShapeDtypeStruct((N, 1), f32),
    )(acc2, s2, dis, b3r, sums3, g3r, be3r, pc, Wci, bcir, Wcl, bclr)

    return out


# speedup vs baseline: 1.0481x; 1.0481x over previous
import functools

import jax
import jax.numpy as jnp
from jax import lax
from jax.experimental import pallas as pl
from jax.experimental.pallas import tpu as pltpu
from jax.experimental.pallas import tpu_sc as plsc

N = 50000
E = 800000
ROWS = E // 128
NSC = 2
NTILE = 16
NROW = N // NTILE
R_TC = 5000
GRID = N // R_TC

_MESH = dict(core_axis_name="c", subcore_axis_name="s")


def _deg_body(dst_hbm, out_hbm, idx_v, part_v, ones_v):
    cid = lax.axis_index("c")
    sid = lax.axis_index("s")
    wid = sid * NSC + cid
    per = E // (NSC * NTILE)
    nfull = per // 16
    rem = per - nfull * 16

    def zbody(i, _):
        part_v[pl.ds(i * 16, 16)] = jnp.zeros((16,), jnp.float32)
        return _
    lax.fori_loop(0, N // 16, zbody, 0)
    idx_v[pl.ds(per, 16)] = jnp.zeros((16,), jnp.int32)
    pltpu.sync_copy(dst_hbm.at[pl.ds(wid * per, per)], idx_v.at[pl.ds(0, per)])

    def body(i, _):
        iv = idx_v[pl.ds(i * 16, 16)]
        plsc.addupdate_scatter(part_v, [iv], ones_v[...])
        return _
    lax.fori_loop(0, nfull, body, 0)
    iv = idx_v[pl.ds(nfull * 16, 16)]
    mask = lax.iota(jnp.int32, 16) < rem
    plsc.addupdate_scatter(part_v, [iv], ones_v[...], mask=mask)
    pltpu.sync_copy(part_v, out_hbm.at[wid])


@functools.partial(
    pl.kernel,
    out_type=jax.ShapeDtypeStruct((NSC * NTILE, N), jnp.float32),
    mesh=plsc.VectorSubcoreMesh(**_MESH),
    compiler_params=pltpu.CompilerParams(needs_layout_passes=False, use_tc_tiling_on_sc=False),
    scratch_types=[
        pltpu.VMEM((E // (NSC * NTILE) + 16,), jnp.int32),
        pltpu.VMEM((N,), jnp.float32),
        pltpu.VMEM((16,), jnp.float32),
    ],
)
def _deg_kernel(dst_hbm, out_hbm, idx_v, part_v, ones_v):
    ones_v[...] = jnp.ones((16,), jnp.float32)
    _deg_body(dst_hbm, out_hbm, idx_v, part_v, ones_v)


def _make_prop(D, split_edges):
    C = 64 if D == 16 else 32
    NB = 10 if D == 16 else 5

    def body(tab_hbm, src_hbm, dst_hbm, out_hbm, *refs):
        src_v, dst_v, srow_v, drow_v = refs[:4]
        rbufs = refs[4:4 + NB]
        acc_sh = refs[4 + NB]
        sems = refs[5 + NB:]
        zbuf = rbufs[0]
        cid = lax.axis_index("c")
        sid = lax.axis_index("s")

        def zb(i, _):
            for k in range(D // 16):
                zbuf[i, pl.ds(k * 16, 16)] = jnp.zeros((16,), jnp.float32)
            return _
        lax.fori_loop(0, 128, zb, 0)
        zbuf2d = zbuf
        base = sid * NROW
        nz = NROW // 128
        rz = NROW - nz * 128

        def zcopy(i, _):
            pltpu.sync_copy(zbuf2d, acc_sh.at[pl.ds(base + i * 128, 128)])
            return _
        lax.fori_loop(0, nz, zcopy, 0)
        if rz:
            pltpu.sync_copy(zbuf2d.at[pl.ds(0, rz)],
                            acc_sh.at[pl.ds(base + nz * 128, rz)])
        plsc.subcore_barrier()

        if split_edges:
            rows_sc = ROWS // NSC
            base_sc = cid * rows_sc
        else:
            rows_sc = ROWS
            base_sc = jnp.int32(0)
        q, r = rows_sc // NTILE, rows_sc % NTILE
        start = base_sc + sid * q + jnp.minimum(sid, r)
        count = q + (sid < r).astype(jnp.int32)

        nch = count // C
        rem = count - nch * C

        def chunk(g, _):
            r0 = start + g * C
            pltpu.sync_copy(src_hbm.at[pl.ds(r0, C)], src_v)
            pltpu.sync_copy(dst_hbm.at[pl.ds(r0, C)], dst_v)
            descs = [pltpu.async_copy(tab_hbm.at[cid].at[src_v.at[j]],
                                      rbufs[j], sems[j])
                     for j in range(NB)]
            for j in range(C):
                b = j % NB
                descs[b].wait()
                pltpu.sync_copy(rbufs[b], acc_sh.at[dst_v.at[j]], add=True)
                if j + NB < C:
                    descs[b] = pltpu.async_copy(
                        tab_hbm.at[cid].at[src_v.at[j + NB]], rbufs[b],
                        sems[b])
            return _
        lax.fori_loop(0, nch, chunk, 0)

        def tail(t, _):
            r0 = start + nch * C + t
            pltpu.sync_copy(src_hbm.at[r0], srow_v)
            pltpu.sync_copy(dst_hbm.at[r0], drow_v)
            pltpu.async_copy(tab_hbm.at[cid].at[srow_v], rbufs[0],
                             sems[0]).wait()
            pltpu.sync_copy(rbufs[0], acc_sh.at[drow_v], add=True)
            return _
        lax.fori_loop(0, rem, tail, 0)

        plsc.subcore_barrier()
        pltpu.sync_copy(acc_sh.at[pl.ds(base, NROW)],
                        out_hbm.at[cid, pl.ds(base, NROW)])

    kern = pl.kernel(
        body,
        out_type=jax.ShapeDtypeStruct((NSC, N, D), jnp.float32),
        mesh=plsc.VectorSubcoreMesh(**_MESH),
        compiler_params=pltpu.CompilerParams(needs_layout_passes=False, use_tc_tiling_on_sc=False),
        scratch_types=(
            [pltpu.VMEM((C, 128), jnp.int32),
             pltpu.VMEM((C, 128), jnp.int32),
             pltpu.VMEM((128,), jnp.int32),
             pltpu.VMEM((128,), jnp.int32)]
            + [pltpu.VMEM((128, D), jnp.float32) for _ in range(NB)]
            + [pltpu.VMEM_SHARED((N, D), jnp.float32)]
            + [pltpu.SemaphoreType.DMA for _ in range(NB)]
        ),
    )
    return kern


def _const(shape):
    return pl.BlockSpec(shape, lambda i: tuple(0 for _ in shape))


def _prep_body(degp, x, Wc1, bc1, Wc2, bc2, dis_o, s0_o, pc_o):
    deg = 1.0 + jnp.sum(degp[...], axis=1, keepdims=True)
    dis = lax.rsqrt(deg)
    dis_o[...] = dis
    xb = x[...]
    s = dis * xb
    s = jnp.pad(s, ((0, 0), (0, 6)))
    s0_o[0] = s
    s0_o[1] = s
    charges = xb[:, 8:9]
    h = jax.nn.relu(charges * Wc1[...] + bc1[...])
    pc_o[...] = jnp.dot(h, Wc2[...]) + bc2[...]


def _prep_call(deg_parts, x, Wc1, bc1, Wc2, bc2):
    return pl.pallas_call(
        _prep_body,
        grid=(GRID,),
        in_specs=[
            pl.BlockSpec((R_TC, NSC * NTILE), lambda i: (i, 0)),
            pl.BlockSpec((R_TC, 10), lambda i: (i, 0)),
            _const((1, 32)), _const((1, 32)),
            _const((32, 16)), _const((1, 16)),
        ],
        out_specs=[
            pl.BlockSpec((R_TC, 1), lambda i: (i, 0)),
            pl.BlockSpec((NSC, R_TC, 16), lambda i: (0, i, 0)),
            pl.BlockSpec((R_TC, 16), lambda i: (i, 0)),
        ],
        out_shape=[
            jax.ShapeDtypeStruct((N, 1), jnp.float32),
            jax.ShapeDtypeStruct((NSC, N, 16), jnp.float32),
            jax.ShapeDtypeStruct((N, 16), jnp.float32),
        ],
    )(deg_parts, x, Wc1, bc1, Wc2, bc2)


def _z1(acc, s0, dis, W1p, b1):
    p = dis[...] * (acc[0] + acc[1] + s0[0])
    return jnp.dot(p, W1p[...]) + b1[...]


def _z2(acc, s1, dis, b2):
    zl = dis[...] * (acc[0] + s1[0])
    zr = dis[...] * (acc[1] + s1[1])
    return jnp.concatenate([zl, zr], axis=1) + b2[...]


def _z3(acc, s2, dis, b3):
    return dis[...] * (acc[0] + acc[1] + s2[0]) + b3[...]


def _bn_relu(z, sums, g, b):
    mu = sums[0:1] * (1.0 / N)
    var = sums[1:2] * (1.0 / N) - mu * mu
    return jax.nn.relu(g * (z - mu) * lax.rsqrt(var + 1e-5) + b)


def _sums_body(zfn, nz_in, *refs):
    ins = refs[:nz_in]
    sums_o = refs[nz_in]
    accv = refs[nz_in + 1]
    z = zfn(*ins)
    i = pl.program_id(0)

    @pl.when(i == 0)
    def _():
        accv[...] = jnp.zeros_like(accv)

    accv[0:1] += jnp.sum(z, axis=0, keepdims=True)
    accv[1:2] += jnp.sum(z * z, axis=0, keepdims=True)

    @pl.when(i == GRID - 1)
    def _():
        sums_o[...] = accv[...]


def _layer_sums(zfn, nz_in, D, in_arrays, in_specs):
    return pl.pallas_call(
        functools.partial(_sums_body, zfn, nz_in),
        grid=(GRID,),
        in_specs=in_specs,
        out_specs=pl.BlockSpec((2, D), lambda i: (0, 0)),
        out_shape=jax.ShapeDtypeStruct((2, D), jnp.float32),
        scratch_shapes=[pltpu.VMEM((2, D), jnp.float32)],
    )(*in_arrays)


def kernel(x, edge_index, Wc1, bc1, Wc2, bc2, W1, b1, g1, be1,
           W2, b2, g2, be2, W3, b3, g3, be3, Wci, bci, Wcl, bcl):
    f32 = jnp.float32
    src2d = edge_index[0].reshape(ROWS, 128)
    dst2d = edge_index[1].reshape(ROWS, 128)
    W1p = jnp.pad(W1, ((0, 6), (0, 0)))
    r2 = lambda v: v.reshape(1, -1)
    bc1r, bc2r, b1r, g1r, be1r = map(r2, (bc1, bc2, b1, g1, be1))
    b2r, g2r, be2r, b3r, g3r, be3r = map(r2, (b2, g2, be2, b3, g3, be3))
    bcir, bclr = r2(bci), r2(bcl)

    deg_parts = _deg_kernel(edge_index[1]).T
    dis, s0, pc = _prep_call(deg_parts, x, Wc1.reshape(1, 32), bc1r,
                             Wc2, bc2r)

    prop16 = _make_prop(16, True)
    prop32e = _make_prop(32, True)
    prop32c = _make_prop(32, False)

    acc0 = prop16(s0, src2d, dst2d)

    spec_dis = pl.BlockSpec((R_TC, 1), lambda i: (i, 0))
    spec_p16 = pl.BlockSpec((NSC, R_TC, 16), lambda i: (0, i, 0))
    spec_p32 = pl.BlockSpec((NSC, R_TC, 32), lambda i: (0, i, 0))
    spec_s16 = pl.BlockSpec((1, R_TC, 16), lambda i: (0, i, 0))
    spec_s32 = pl.BlockSpec((1, R_TC, 32), lambda i: (0, i, 0))

    sums1 = _layer_sums(
        _z1, 5, 128,
        [acc0, s0, dis, W1p, b1r],
        [spec_p16, spec_s16, spec_dis, _const((16, 128)), _const((1, 128))])

    def _l1b(acc, s0r, disr, W1r, b1rr, sums, g, be, W2r, s1_o):
        z = _z1(acc, s0r, disr, W1r, b1rr)
        h = _bn_relu(z, sums[...], g[...], be[...])
        t = disr[...] * jnp.dot(h, W2r[...])
        s1_o[0] = t[:, :32]
        s1_o[1] = t[:, 32:]

    s1 = pl.pallas_call(
        _l1b,
        grid=(GRID,),
        in_specs=[spec_p16, spec_s16, spec_dis, _const((16, 128)),
                  _const((1, 128)), _const((2, 128)), _const((1, 128)),
                  _const((1, 128)), _const((128, 64))],
        out_specs=pl.BlockSpec((NSC, R_TC, 32), lambda i: (0, i, 0)),
        out_shape=jax.ShapeDtypeStruct((NSC, N, 32), f32),
    )(acc0, s0, dis, W1p, b1r, sums1, g1r, be1r, W2)

    acc1 = prop32c(s1, src2d, dst2d)

    sums2 = _layer_sums(
        _z2, 4, 64,
        [acc1, s1, dis, b2r],
        [spec_p32, spec_p32, spec_dis, _const((1, 64))])

    def _l2b(acc, s1r, disr, b2rr, sums, g, be, W3r, s2_o):
        z = _z2(acc, s1r, disr, b2rr)
        h = _bn_relu(z, sums[...], g[...], be[...])
        t = disr[...] * jnp.dot(h, W3r[...])
        s2_o[0] = t
        s2_o[1] = t

    s2 = pl.pallas_call(
        _l2b,
        grid=(GRID,),
        in_specs=[spec_p32, spec_p32, spec_dis, _const((1, 64)),
                  _const((2, 64)), _const((1, 64)), _const((1, 64)),
                  _const((64, 32))],
        out_specs=pl.BlockSpec((NSC, R_TC, 32), lambda i: (0, i, 0)),
        out_shape=jax.ShapeDtypeStruct((NSC, N, 32), f32),
    )(acc1, s1, dis, b2r, sums2, g2r, be2r, W3)

    acc2 = prop32e(s2, src2d, dst2d)

    sums3 = _layer_sums(
        _z3, 4, 32,
        [acc2, s2, dis, b3r],
        [spec_p32, spec_s32, spec_dis, _const((1, 32))])

    def _l3b(acc, s2r, disr, b3rr, sums, g, be, pcr, Wcir, bcirr, Wclr,
             bclrr, out_o):
        z = _z3(acc, s2r, disr, b3rr)
        h = _bn_relu(z, sums[...], g[...], be[...])
        hc = jnp.concatenate([h, pcr[...]], axis=1)
        hh = jax.nn.relu(jnp.dot(hc, Wcir[...]) + bcirr[...])
        out_o[...] = jax.nn.sigmoid(jnp.dot(hh, Wclr[...]) + bclrr[...])

    out = pl.pallas_call(
        _l3b,
        grid=(GRID,),
        in_specs=[spec_p32, spec_s32, spec_dis, _const((1, 32)),
                  _const((2, 32)), _const((1, 32)), _const((1, 32)),
                  pl.BlockSpec((R_TC, 16), lambda i: (i, 0)),
                  _const((48, 32)), _const((1, 32)),
                  _const((32, 1)), _const((1, 1))],
        out_specs=pl.BlockSpec((R_TC, 1), lambda i: (i, 0)),
        out_shape=jax.ShapeDtypeStruct((N, 1), f32),
    )(acc2, s2, dis, b3r, sums3, g3r, be3r, pc, Wci, bcir, Wcl, bclr)

    return out
